# baseline (speedup 1.0000x reference)
import jax
import jax.numpy as jnp
from jax.experimental import pallas as pl
from jax.experimental.pallas import tpu as pltpu

IN_DIMS = 1024
SEQ_LEN = 2048
TOP_K = 64
BLK = 256
N_QUAT = 5
SUB = 4
SCALE = 16.0


def _process(scores, xf, w, b):
    lo = jnp.full((1, BLK), -1.01 * SCALE * SCALE, jnp.float32)
    width = jnp.full((1, BLK), 2.02 * SCALE * SCALE, jnp.float32)
    for _ in range(N_QUAT):
        q = width * 0.25
        t1 = lo + q
        t2 = lo + 2.0 * q
        t3 = lo + 3.0 * q
        c1 = jnp.sum((scores >= t1).astype(jnp.float32), axis=0,
                     keepdims=True)
        c2 = jnp.sum((scores >= t2).astype(jnp.float32), axis=0,
                     keepdims=True)
        c3 = jnp.sum((scores >= t3).astype(jnp.float32), axis=0,
                     keepdims=True)
        lo = jnp.where(c3 >= TOP_K, t3,
                       jnp.where(c2 >= TOP_K, t2,
                                 jnp.where(c1 >= TOP_K, t1, lo)))
        width = q

    e = jnp.exp(scores * (1.0 / (SCALE * SCALE)))
    a = jnp.where(scores >= lo, e, 1.0)
    denom = jnp.sum(a, axis=0, keepdims=True)
    attn = (a * (1.0 / denom)).astype(jnp.bfloat16)

    g = jax.lax.dot_general(
        attn, xf, (((0,), (0,)), ((), ())),
        preferred_element_type=jnp.float32)
    out = jax.lax.dot_general(
        g.astype(jnp.bfloat16), w, (((1,), (1,)), ((), ())),
        preferred_element_type=jnp.float32)
    return out + b


def _fused_body(xf_ref, w_ref, b_ref, out_ref, xn_ref):
    j = pl.program_id(1)

    @pl.when(j == 0)
    def _init():
        xf32 = xf_ref[0].astype(jnp.float32)
        rn = jax.lax.rsqrt(jnp.maximum(
            jnp.sum(xf32 * xf32, axis=1, keepdims=True), 1e-24))
        xn_ref[...] = (xf32 * (rn * SCALE)).astype(jnp.float8_e4m3fn)

    xn = xn_ref[...]
    base = j * (SUB * BLK)
    scores = [
        jax.lax.dot_general(
            xn, xn_ref[pl.ds(base + k * BLK, BLK), :],
            (((1,), (1,)), ((), ())),
            preferred_element_type=jnp.float32)
        for k in range(SUB)
    ]

    xf = xf_ref[0]
    w = w_ref[...]
    b = b_ref[...]
    for k in range(SUB):
        out_ref[0, k * BLK:(k + 1) * BLK, :] = _process(scores[k], xf, w, b)


def kernel(x, W, b):
    B, S, D = x.shape
    b2 = b.reshape(1, D)
    xb16 = x.astype(jnp.bfloat16)
    Wb16 = W.astype(jnp.bfloat16)
    out = pl.pallas_call(
        _fused_body,
        grid=(B, S // (SUB * BLK)),
        in_specs=[
            pl.BlockSpec((1, S, D), lambda bi, j: (bi, 0, 0)),
            pl.BlockSpec((D, D), lambda bi, j: (0, 0)),
            pl.BlockSpec((1, D), lambda bi, j: (0, 0)),
        ],
        out_specs=pl.BlockSpec((1, SUB * BLK, D), lambda bi, j: (bi, j, 0)),
        out_shape=jax.ShapeDtypeStruct((B, S, D), jnp.float32),
        scratch_shapes=[
            pltpu.VMEM((S, D), jnp.float8_e4m3fn),
        ],
    )(xb16, Wb16, b2)
    return out


# speedup vs baseline: 1.1250x; 1.1250x over previous
import jax
import jax.numpy as jnp
from jax.experimental import pallas as pl
from jax.experimental.pallas import tpu as pltpu

IN_DIMS = 1024
SEQ_LEN = 2048
TOP_K = 64
BLK = 256
N_BISECT = 10
SUB = 4
SCALE = 16.0


def _process(scores, xf, w, b):
    lo = jnp.full((1, BLK), -1.01 * SCALE * SCALE, jnp.float32)
    hi = jnp.full((1, BLK), 1.01 * SCALE * SCALE, jnp.float32)
    for _ in range(N_BISECT):
        mid = 0.5 * (lo + hi)
        cnt = jnp.sum((scores >= mid).astype(jnp.float32), axis=0,
                      keepdims=True)
        ge = cnt >= TOP_K
        lo, hi = jnp.where(ge, mid, lo), jnp.where(ge, hi, mid)

    e = jnp.exp(scores * (1.0 / (SCALE * SCALE)))
    a = jnp.where(scores >= lo, e, 1.0)
    denom = jnp.sum(a, axis=0, keepdims=True)
    attn = (a * (1.0 / denom)).astype(jnp.bfloat16)

    g = jax.lax.dot_general(
        attn, xf, (((0,), (0,)), ((), ())),
        preferred_element_type=jnp.float32)
    out = jax.lax.dot_general(
        g.astype(jnp.bfloat16), w, (((1,), (1,)), ((), ())),
        preferred_element_type=jnp.float32)
    return out + b


def _fused_body(xf_ref, w_ref, b_ref, out_ref, xn_ref):
    j = pl.program_id(1)

    @pl.when(j == 0)
    def _init():
        xf32 = xf_ref[0].astype(jnp.float32)
        rn = jax.lax.rsqrt(jnp.maximum(
            jnp.sum(xf32 * xf32, axis=1, keepdims=True), 1e-24))
        xn_ref[...] = (xf32 * (rn * SCALE)).astype(jnp.float8_e4m3fn)

    xn = xn_ref[...]
    base = j * (SUB * BLK)
    scores = [
        jax.lax.dot_general(
            xn, xn_ref[pl.ds(base + k * BLK, BLK), :],
            (((1,), (1,)), ((), ())),
            preferred_element_type=jnp.float32)
        for k in range(SUB)
    ]

    xf = xf_ref[0]
    w = w_ref[...]
    b = b_ref[...]
    for k in range(SUB):
        out_ref[0, k * BLK:(k + 1) * BLK, :] = _process(scores[k], xf, w, b)


def kernel(x, W, b):
    B, S, D = x.shape
    b2 = b.reshape(1, D)
    xb16 = x.astype(jnp.bfloat16)
    Wb16 = W.astype(jnp.bfloat16)
    out = pl.pallas_call(
        _fused_body,
        grid=(B, S // (SUB * BLK)),
        in_specs=[
            pl.BlockSpec((1, S, D), lambda bi, j: (bi, 0, 0)),
            pl.BlockSpec((D, D), lambda bi, j: (0, 0)),
            pl.BlockSpec((1, D), lambda bi, j: (0, 0)),
        ],
        out_specs=pl.BlockSpec((1, SUB * BLK, D), lambda bi, j: (bi, j, 0)),
        out_shape=jax.ShapeDtypeStruct((B, S, D), jnp.float32),
        scratch_shapes=[
            pltpu.VMEM((S, D), jnp.float8_e4m3fn),
        ],
    )(xb16, Wb16, b2)
    return out


# speedup vs baseline: 1.2150x; 1.0800x over previous
import jax
import jax.numpy as jnp
from jax.experimental import pallas as pl
from jax.experimental.pallas import tpu as pltpu

IN_DIMS = 1024
SEQ_LEN = 2048
TOP_K = 64
BLK = 256
N_BISECT = 8
SUB = 4
SCALE = 16.0


def _process(scores, xf, w, b):
    lo = jnp.full((1, BLK), -1.01 * SCALE * SCALE, jnp.float32)
    hi = jnp.full((1, BLK), 1.01 * SCALE * SCALE, jnp.float32)
    for _ in range(N_BISECT):
        mid = 0.5 * (lo + hi)
        cnt = jnp.sum((scores >= mid).astype(jnp.float32), axis=0,
                      keepdims=True)
        ge = cnt >= TOP_K
        lo, hi = jnp.where(ge, mid, lo), jnp.where(ge, hi, mid)

    e = jnp.exp(scores * (1.0 / (SCALE * SCALE)))
    a = jnp.where(scores >= lo, e, 1.0)
    denom = jnp.sum(a, axis=0, keepdims=True)
    attn = (a * (1.0 / denom)).astype(jnp.bfloat16)

    g = jax.lax.dot_general(
        attn, xf, (((0,), (0,)), ((), ())),
        preferred_element_type=jnp.float32)
    out = jax.lax.dot_general(
        g.astype(jnp.bfloat16), w, (((1,), (1,)), ((), ())),
        preferred_element_type=jnp.float32)
    return out + b


def _fused_body(xf_ref, w_ref, b_ref, out_ref, xn_ref):
    j = pl.program_id(1)

    @pl.when(j == 0)
    def _init():
        xf32 = xf_ref[0].astype(jnp.float32)
        rn = jax.lax.rsqrt(jnp.maximum(
            jnp.sum(xf32 * xf32, axis=1, keepdims=True), 1e-24))
        xn_ref[...] = (xf32 * (rn * SCALE)).astype(jnp.float8_e4m3fn)

    xn = xn_ref[...]
    base = j * (SUB * BLK)
    scores = [
        jax.lax.dot_general(
            xn, xn_ref[pl.ds(base + k * BLK, BLK), :],
            (((1,), (1,)), ((), ())),
            preferred_element_type=jnp.float32)
        for k in range(SUB)
    ]

    xf = xf_ref[0]
    w = w_ref[...]
    b = b_ref[...]
    for k in range(SUB):
        out_ref[0, k * BLK:(k + 1) * BLK, :] = _process(scores[k], xf, w, b)


def kernel(x, W, b):
    B, S, D = x.shape
    b2 = b.reshape(1, D)
    xb16 = x.astype(jnp.bfloat16)
    Wb16 = W.astype(jnp.bfloat16)
    out = pl.pallas_call(
        _fused_body,
        grid=(B, S // (SUB * BLK)),
        in_specs=[
            pl.BlockSpec((1, S, D), lambda bi, j: (bi, 0, 0)),
            pl.BlockSpec((D, D), lambda bi, j: (0, 0)),
            pl.BlockSpec((1, D), lambda bi, j: (0, 0)),
        ],
        out_specs=pl.BlockSpec((1, SUB * BLK, D), lambda bi, j: (bi, j, 0)),
        out_shape=jax.ShapeDtypeStruct((B, S, D), jnp.float32),
        scratch_shapes=[
            pltpu.VMEM((S, D), jnp.float8_e4m3fn),
        ],
    )(xb16, Wb16, b2)
    return out


# speedup vs baseline: 1.2466x; 1.0260x over previous
import jax
import jax.numpy as jnp
from jax.experimental import pallas as pl
from jax.experimental.pallas import tpu as pltpu

IN_DIMS = 1024
SEQ_LEN = 2048
TOP_K = 64
BLK = 256
N_BISECT = 8
SUB = 8
SCALE = 16.0


def _process(scores, xf, w, b):
    lo = jnp.full((1, BLK), -1.01 * SCALE * SCALE, jnp.float32)
    hi = jnp.full((1, BLK), 1.01 * SCALE * SCALE, jnp.float32)
    for _ in range(N_BISECT):
        mid = 0.5 * (lo + hi)
        cnt = jnp.sum((scores >= mid).astype(jnp.float32), axis=0,
                      keepdims=True)
        ge = cnt >= TOP_K
        lo, hi = jnp.where(ge, mid, lo), jnp.where(ge, hi, mid)

    e = jnp.exp(scores * (1.0 / (SCALE * SCALE)))
    a = jnp.where(scores >= lo, e, 1.0)
    denom = jnp.sum(a, axis=0, keepdims=True)
    attn = (a * (1.0 / denom)).astype(jnp.bfloat16)

    g = jax.lax.dot_general(
        attn, xf, (((0,), (0,)), ((), ())),
        preferred_element_type=jnp.float32)
    out = jax.lax.dot_general(
        g.astype(jnp.bfloat16), w, (((1,), (1,)), ((), ())),
        preferred_element_type=jnp.float32)
    return out + b


def _fused_body(xf_ref, w_ref, b_ref, out_ref, xn_ref):
    j = pl.program_id(1)

    @pl.when(j == 0)
    def _init():
        xf32 = xf_ref[0].astype(jnp.float32)
        rn = jax.lax.rsqrt(jnp.maximum(
            jnp.sum(xf32 * xf32, axis=1, keepdims=True), 1e-24))
        xn_ref[...] = (xf32 * (rn * SCALE)).astype(jnp.float8_e4m3fn)

    xn = xn_ref[...]
    base = j * (SUB * BLK)
    scores = [
        jax.lax.dot_general(
            xn, xn_ref[pl.ds(base + k * BLK, BLK), :],
            (((1,), (1,)), ((), ())),
            preferred_element_type=jnp.float32)
        for k in range(SUB)
    ]

    xf = xf_ref[0]
    w = w_ref[...]
    b = b_ref[...]
    for k in range(SUB):
        out_ref[0, k * BLK:(k + 1) * BLK, :] = _process(scores[k], xf, w, b)


def kernel(x, W, b):
    B, S, D = x.shape
    b2 = b.reshape(1, D)
    xb16 = x.astype(jnp.bfloat16)
    Wb16 = W.astype(jnp.bfloat16)
    out = pl.pallas_call(
        _fused_body,
        grid=(B, S // (SUB * BLK)),
        in_specs=[
            pl.BlockSpec((1, S, D), lambda bi, j: (bi, 0, 0)),
            pl.BlockSpec((D, D), lambda bi, j: (0, 0)),
            pl.BlockSpec((1, D), lambda bi, j: (0, 0)),
        ],
        out_specs=pl.BlockSpec((1, SUB * BLK, D), lambda bi, j: (bi, j, 0)),
        out_shape=jax.ShapeDtypeStruct((B, S, D), jnp.float32),
        scratch_shapes=[
            pltpu.VMEM((S, D), jnp.float8_e4m3fn),
        ],
    )(xb16, Wb16, b2)
    return out
